# baseline (speedup 1.0000x reference)
import jax
import jax.numpy as jnp
from jax.experimental import pallas as pl
from jax.experimental.pallas import tpu as pltpu

N0, N1, EMB = 384, 384, 96
BI = 16
NBUF = 6
STEPS = N0 // BI


def _body(e0_ref, e1_ref, out_hbm, buf_ref, sems):
    g = pl.program_id(0)
    slot = jax.lax.rem(g, NBUF)

    def _copy(step, s):
        return pltpu.make_async_copy(
            buf_ref.at[s], out_hbm.at[pl.ds(step * BI, BI)], sems.at[s]
        )

    @pl.when(g >= NBUF)
    def _():
        _copy(g - NBUF, slot).wait()

    e1 = e1_ref[...]
    for b in range(BI):
        buf_ref[slot, b] = e1 + e0_ref[b]

    _copy(g, slot).start()

    @pl.when(g == STEPS - 1)
    def _():
        for step in range(max(STEPS - NBUF, 0), STEPS):
            _copy(step, step % NBUF).wait()


def kernel(x, emb_0, emb_1):
    del x
    e1t = emb_1.T
    e0c = emb_0[:, :, None]
    out3 = pl.pallas_call(
        _body,
        grid=(STEPS,),
        in_specs=[
            pl.BlockSpec((BI, EMB, 1), lambda g: (g, 0, 0)),
            pl.BlockSpec((EMB, N1), lambda g: (0, 0)),
        ],
        out_specs=pl.BlockSpec(memory_space=pl.ANY),
        out_shape=jax.ShapeDtypeStruct((N0, EMB, N1), jnp.float32),
        scratch_shapes=[
            pltpu.VMEM((NBUF, BI, EMB, N1), jnp.float32),
            pltpu.SemaphoreType.DMA((NBUF,)),
        ],
    )(e0c, e1t)
    return out3.transpose(0, 2, 1)[None]
